# baseline (speedup 1.0000x reference)
import functools

import jax
import jax.numpy as jnp
from jax import lax
from jax.experimental import pallas as pl
from jax.experimental.pallas import tpu as pltpu
from jax.experimental.pallas import tpu_sc as plsc

_L = 16
_NB = 512
_IMIN = -2**31 + 1


def _scores_body(x_ref, r_ref, o_ref):
    o_ref[...] = lax.dot_general(
        r_ref[...], x_ref[...], dimension_numbers=(((0,), (1,)), ((), ())),
        preferred_element_type=jnp.float32)


def _zero_body(o_ref):
    o_ref[...] = jnp.zeros_like(o_ref)


def _ffn_body(out_in_ref, idx_ref, xg_ref, gates_ref, wi0_ref, wi1_ref,
              ws0_ref, ws1_ref, wo0_ref, wo1_ref, out_ref,
              xs_ref, h_ref, racc_ref, rd_sem, wr_sem, *, A, G, E):
    e = pl.program_id(0)
    DH = wi0_ref.shape[1]
    FH = wo0_ref.shape[1]

    @pl.when(e > 0)
    def _fence():
        for g in range(G):
            t = idx_ref[(e - 1) * G + g]
            pltpu.make_async_copy(
                racc_ref.at[pl.ds(g, 1), :], out_ref.at[pl.ds(t, 1), :], wr_sem
            ).wait()

    for g in range(G):
        t = idx_ref[e * G + g]
        pltpu.make_async_copy(
            out_ref.at[pl.ds(t, 1), :], racc_ref.at[pl.ds(g, 1), :], rd_sem
        ).start()

    gcol = jnp.transpose(gates_ref[0])
    xsv = xg_ref[0] * gcol
    xs_ref[...] = xsv
    x0, x1 = xsv[:, :DH], xs_ref[:, DH:]
    mi = (jnp.dot(x0, wi0_ref[0], preferred_element_type=jnp.float32)
          + jnp.dot(x1, wi1_ref[0], preferred_element_type=jnp.float32))
    sw = (jnp.dot(x0, ws0_ref[0], preferred_element_type=jnp.float32)
          + jnp.dot(x1, ws1_ref[0], preferred_element_type=jnp.float32))
    h_ref[...] = (mi * jax.nn.sigmoid(mi)) * sw
    o = (jnp.dot(h_ref[:, :FH], wo0_ref[0], preferred_element_type=jnp.float32)
         + jnp.dot(h_ref[:, FH:], wo1_ref[0],
                   preferred_element_type=jnp.float32))

    for g in range(G):
        t = idx_ref[e * G + g]
        pltpu.make_async_copy(
            out_ref.at[pl.ds(t, 1), :], racc_ref.at[pl.ds(g, 1), :], rd_sem
        ).wait()
    racc_ref[...] += o
    for g in range(G):
        t = idx_ref[e * G + g]
        pltpu.make_async_copy(
            racc_ref.at[pl.ds(g, 1), :], out_ref.at[pl.ds(t, 1), :], wr_sem
        ).start()

    @pl.when(e == E - 1)
    def _drain():
        for g in range(G):
            t = idx_ref[e * G + g]
            pltpu.make_async_copy(
                racc_ref.at[pl.ds(g, 1), :], out_ref.at[pl.ds(t, 1), :], wr_sem
            ).wait()


def _sc_route_body(scores_hbm, x_hbm, idx_out, gate_out, routed_out,
                   row_v, hist_v, csum_v, ge_v, cpos_v, ckey_v, sel_v, gsc_v,
                   rows_v, sem, *, A, E, G, EPW, e0):
    CHUNKS = A // _L
    NBV = _NB // _L
    lanes = lax.iota(jnp.int32, _L)
    ones = jnp.ones((_L,), jnp.int32)
    lane0 = lanes == 0

    def splat(s):
        return jnp.full((_L,), s, jnp.int32)

    def scal(v):
        return jnp.max(v)

    z16 = jnp.zeros((_L,), jnp.int32)
    alltrue = jnp.full((_L,), True)

    def cnt_splat(m):
        return jnp.broadcast_to(plsc.all_reduce_population_count(m), (_L,))

    wid = lax.axis_index("s") * 2 + lax.axis_index("c")

    for ei in range(EPW):
        e = wid * EPW + ei

        pltpu.sync_copy(scores_hbm.at[e0 + e], row_v)

        def zero_hist():
            @plsc.parallel_loop(0, _L * NBV, unroll=8)
            def _(i):
                hist_v[i // NBV, pl.ds((i % NBV) * _L, _L)] = z16

        def scan_and_find(target):
            def rb(v, c):
                acc = jnp.zeros((_L,), jnp.int32)
                for r in range(_L):
                    acc = acc + hist_v[r, pl.ds(v * _L, _L)]
                csum_v[pl.ds(v * _L, _L)] = acc
                return c
            lax.fori_loop(0, NBV, rb, 0)

            def sb(j, carry):
                v = NBV - 1 - j
                col = csum_v[pl.ds(v * _L, _L)]
                rsuf = plsc.cumsum(lax.rev(col, (0,))) + carry
                ge_v[pl.ds(v * _L, _L)] = lax.rev(rsuf, (0,))
                return scal(rsuf)
            lax.fori_loop(0, NBV, sb, jnp.int32(0))

            def fb(j, carry):
                found, bstar = carry
                v = NBV - 1 - j
                gvr = lax.rev(ge_v[pl.ds(v * _L, _L)], (0,))
                m = gvr >= target
                has = scal(plsc.all_reduce_population_count(m)) > 0
                k = scal(plsc.all_reduce_ffs(m))
                cand_b = v * _L + (_L - 1 - k)
                hit = jnp.logical_and(jnp.logical_not(found), has)
                return (jnp.logical_or(found, has),
                        jnp.where(hit, cand_b, bstar))
            _, bstar = lax.fori_loop(0, NBV, fb, (False, jnp.int32(0)))

            cnt_b = scal(plsc.load_gather(csum_v, [splat(bstar)]))
            ge_b = scal(plsc.load_gather(ge_v, [splat(bstar)]))
            return bstar, ge_b - cnt_b

        zero_hist()

        @plsc.parallel_loop(0, CHUNKS, unroll=4)
        def _(i):
            s = row_v[pl.ds(i * _L, _L)]
            u = plsc.bitcast(s, jnp.int32)
            key = jnp.where(u < 0, u ^ jnp.int32(0x7FFFFFFF), u)
            row_v[pl.ds(i * _L, _L)] = plsc.bitcast(key, jnp.float32)
            b = (key >> 23) + _NB // 2
            plsc.addupdate_scatter(hist_v, [lanes, b], ones)

        bstar, c_above = scan_and_find(G)
        need = G - c_above

        def px(i, carry):
            ogt, oeq = carry
            key = plsc.bitcast(row_v[pl.ds(i * _L, _L)], jnp.int32)
            b = (key >> 23) + _NB // 2
            pos = i * _L + lanes
            m_gt = b > bstar
            m_eq = b == bstar
            cgt = plsc.cumsum(jnp.where(m_gt, 1, 0))
            ceq = plsc.cumsum(jnp.where(m_eq, 1, 0))
            plsc.store_scatter(sel_v, [ogt + cgt - 1], pos, mask=m_gt)
            plsc.store_scatter(cpos_v, [oeq + ceq - 1], pos, mask=m_eq)
            plsc.store_scatter(ckey_v, [oeq + ceq - 1], key, mask=m_eq)
            return ogt + cnt_splat(m_gt), oeq + cnt_splat(m_eq)
        _, oeq_v = plsc.parallel_loop(0, CHUNKS, carry=(z16, z16))(px)
        m_cnt = scal(oeq_v)

        plsc.store_scatter(ckey_v, [m_cnt + lanes], splat(_IMIN), mask=alltrue)

        zero_hist()
        nv = (m_cnt + _L - 1) // _L

        def pa2(v, c):
            k = ckey_v[pl.ds(v * _L, _L)]
            valid = (v * _L + lanes) < m_cnt
            b2 = (k >> 14) & (_NB - 1)
            plsc.addupdate_scatter(hist_v, [lanes, b2], ones, mask=valid)
            return c
        lax.fori_loop(0, nv, pa2, 0)

        b2star, c_above2 = scan_and_find(need)
        need2 = need - c_above2
        base2 = c_above + c_above2

        def px2(v, carry):
            osel, oeq = carry
            k = ckey_v[pl.ds(v * _L, _L)]
            p = cpos_v[pl.ds(v * _L, _L)]
            valid = (v * _L + lanes) < m_cnt
            b2 = (k >> 14) & (_NB - 1)
            m_gt = jnp.logical_and(b2 > b2star, valid)
            m_eq = jnp.logical_and(b2 == b2star, valid)
            cgt = plsc.cumsum(jnp.where(m_gt, 1, 0))
            ceq = plsc.cumsum(jnp.where(m_eq, 1, 0))
            plsc.store_scatter(sel_v, [osel + cgt - 1], p, mask=m_gt)
            plsc.store_scatter(cpos_v, [oeq + ceq - 1], p, mask=m_eq)
            plsc.store_scatter(ckey_v, [oeq + ceq - 1], k, mask=m_eq)
            return osel + cnt_splat(m_gt), oeq + cnt_splat(m_eq)
        _, oeq2_v = lax.fori_loop(0, nv, px2, (splat(c_above), z16))
        m2 = scal(oeq2_v)

        plsc.store_scatter(ckey_v, [m2 + lanes], splat(_IMIN), mask=alltrue)

        nv2 = (m2 + _L - 1) // _L

        def pick(r, c):
            def mx1(v, mm):
                return jnp.maximum(mm, scal(ckey_v[pl.ds(v * _L, _L)]))
            mx = lax.fori_loop(0, nv2, mx1, jnp.int32(_IMIN))

            def fpos(v, carry):
                found, p = carry
                ck = ckey_v[pl.ds(v * _L, _L)]
                m = ck == mx
                has = scal(plsc.all_reduce_population_count(m)) > 0
                k = scal(plsc.all_reduce_ffs(m))
                hit = jnp.logical_and(jnp.logical_not(found), has)
                return (jnp.logical_or(found, has),
                        jnp.where(hit, v * _L + k, p))
            _, p = lax.fori_loop(0, nv2, fpos, (False, jnp.int32(0)))

            tokp = plsc.load_gather(cpos_v, [splat(p)])
            plsc.store_scatter(sel_v, [splat(base2 + r)], tokp, mask=lane0)
            plsc.store_scatter(ckey_v, [splat(p)], splat(_IMIN), mask=lane0)
            return c
        lax.fori_loop(0, need2, pick, 0)

        for v in range(G // _L):
            sidx = sel_v[pl.ds(v * _L, _L)]
            key = plsc.bitcast(plsc.load_gather(row_v, [sidx]), jnp.int32)
            u = jnp.where(key < 0, key ^ jnp.int32(0x7FFFFFFF), key)
            s = plsc.bitcast(u, jnp.float32)
            gsc_v[pl.ds(v * _L, _L)] = 1.0 / (1.0 + jnp.exp(-s))

        pltpu.sync_copy(sel_v, idx_out.at[e])
        pltpu.sync_copy(gsc_v, gate_out.at[e])

        pltpu.async_copy(x_hbm.at[sel_v], rows_v, sem).wait()
        pltpu.sync_copy(rows_v, routed_out.at[pl.ds(e * G, G)])


def kernel(x, router_DE, moe_w_in_eD_F, moe_w_swiglu_eD_F, moe_w_out_eF_D):
    A, D = x.shape
    E = router_DE.shape[1]
    F = moe_w_in_eD_F.shape[1]
    G = int(A * 0.25 / E)
    G += (-G) % 8
    G = min(G, A)
    EG = E * G

    ABLK = 2048
    scores = pl.pallas_call(
        _scores_body,
        grid=(A // ABLK,),
        in_specs=[pl.BlockSpec((ABLK, D), lambda i: (i, 0)),
                  pl.BlockSpec((D, E), lambda i: (0, 0))],
        out_specs=pl.BlockSpec((E, ABLK), lambda i: (0, i)),
        out_shape=jax.ShapeDtypeStruct((E, A), jnp.float32),
    )(x, router_DE)

    NW = 32
    NCH = 2
    NE = E // NCH
    EPW = NE // NW
    mesh = plsc.VectorSubcoreMesh(core_axis_name="c", subcore_axis_name="s")

    def route_chunk(e0):
        r = functools.partial(
            pl.kernel, mesh=mesh,
            out_type=(jax.ShapeDtypeStruct((NE, G), jnp.int32),
                      jax.ShapeDtypeStruct((NE, G), jnp.float32),
                      jax.ShapeDtypeStruct((NE * G, D), jnp.float32)),
            scratch_types=[pltpu.VMEM((A,), jnp.float32),
                           pltpu.VMEM((_L, _NB), jnp.int32),
                           pltpu.VMEM((_NB,), jnp.int32),
                           pltpu.VMEM((_NB,), jnp.int32),
                           pltpu.VMEM((A,), jnp.int32),
                           pltpu.VMEM((A + _L,), jnp.int32),
                           pltpu.VMEM((G,), jnp.int32),
                           pltpu.VMEM((G,), jnp.float32),
                           pltpu.VMEM((G, D), jnp.float32),
                           pltpu.SemaphoreType.DMA],
            compiler_params=pltpu.CompilerParams(needs_layout_passes=False),
        )(functools.partial(_sc_route_body, A=A, E=NE, G=G, EPW=EPW, e0=e0))
        return r(scores, x)

    ZBLK = 2048
    out = pl.pallas_call(
        _zero_body,
        grid=(A // ZBLK,),
        out_specs=pl.BlockSpec((ZBLK, D), lambda i: (i, 0)),
        out_shape=jax.ShapeDtypeStruct((A, D), jnp.float32),
    )()

    DH, FH = D // 2, F // 2
    wi3 = moe_w_in_eD_F.reshape(2 * E, DH, F)
    ws3 = moe_w_swiglu_eD_F.reshape(2 * E, DH, F)
    wo3 = moe_w_out_eF_D.reshape(2 * E, FH, D)
    routed_chunks = [route_chunk(c * NE) for c in range(NCH)]
    for c in range(NCH):
        idx_c, gates_c, routed_c = routed_chunks[c]
        s0 = 2 * c * NE
        out = pl.pallas_call(
            functools.partial(_ffn_body, A=A, G=G, E=NE),
            grid=(NE,),
            in_specs=[
                pl.BlockSpec(memory_space=pl.ANY),
                pl.BlockSpec(memory_space=pltpu.SMEM),
                pl.BlockSpec((1, G, D), lambda e: (e, 0, 0)),
                pl.BlockSpec((1, 1, G), lambda e: (e, 0, 0)),
                pl.BlockSpec((1, DH, F), lambda e, s0=s0: (s0 + 2 * e, 0, 0)),
                pl.BlockSpec((1, DH, F),
                             lambda e, s0=s0: (s0 + 2 * e + 1, 0, 0)),
                pl.BlockSpec((1, DH, F), lambda e, s0=s0: (s0 + 2 * e, 0, 0)),
                pl.BlockSpec((1, DH, F),
                             lambda e, s0=s0: (s0 + 2 * e + 1, 0, 0)),
                pl.BlockSpec((1, FH, D), lambda e, s0=s0: (s0 + 2 * e, 0, 0)),
                pl.BlockSpec((1, FH, D),
                             lambda e, s0=s0: (s0 + 2 * e + 1, 0, 0)),
            ],
            out_specs=pl.BlockSpec(memory_space=pl.ANY),
            out_shape=jax.ShapeDtypeStruct((A, D), jnp.float32),
            scratch_shapes=[pltpu.VMEM((G, D), jnp.float32),
                            pltpu.VMEM((G, F), jnp.float32),
                            pltpu.VMEM((G, D), jnp.float32),
                            pltpu.SemaphoreType.DMA,
                            pltpu.SemaphoreType.DMA],
            input_output_aliases={0: 0},
        )(out, idx_c.reshape(-1), routed_c.reshape(NE, G, D),
          gates_c.reshape(NE, 1, G), wi3, wi3, ws3, ws3, wo3, wo3)
    return out


# speedup vs baseline: 1.0167x; 1.0167x over previous
import functools

import jax
import jax.numpy as jnp
from jax import lax
from jax.experimental import pallas as pl
from jax.experimental.pallas import tpu as pltpu
from jax.experimental.pallas import tpu_sc as plsc

_L = 16
_NB = 512
_IMIN = -2**31 + 1


def _scores_body(x_ref, r_ref, o_ref):
    o_ref[...] = lax.dot_general(
        r_ref[...], x_ref[...], dimension_numbers=(((0,), (1,)), ((), ())),
        preferred_element_type=jnp.float32)


def _zero_body(o_ref):
    o_ref[...] = jnp.zeros_like(o_ref)


def _ffn_body(out_in_ref, idx_ref, xg_ref, gates_ref, wi0_ref, wi1_ref,
              ws0_ref, ws1_ref, wo0_ref, wo1_ref, out_ref,
              xs_ref, h_ref, racc_ref, rd_sem, wr_sem, *, A, G, E):
    e = pl.program_id(0)
    DH = wi0_ref.shape[1]
    FH = wo0_ref.shape[1]

    @pl.when(e > 0)
    def _fence():
        for g in range(G):
            t = idx_ref[(e - 1) * G + g]
            pltpu.make_async_copy(
                racc_ref.at[pl.ds(g, 1), :], out_ref.at[pl.ds(t, 1), :], wr_sem
            ).wait()

    for g in range(G):
        t = idx_ref[e * G + g]
        pltpu.make_async_copy(
            out_ref.at[pl.ds(t, 1), :], racc_ref.at[pl.ds(g, 1), :], rd_sem
        ).start()

    gcol = jnp.transpose(gates_ref[0])
    xsv = xg_ref[0] * gcol
    xs_ref[...] = xsv
    x0, x1 = xsv[:, :DH], xs_ref[:, DH:]
    mi = (jnp.dot(x0, wi0_ref[0], preferred_element_type=jnp.float32)
          + jnp.dot(x1, wi1_ref[0], preferred_element_type=jnp.float32))
    sw = (jnp.dot(x0, ws0_ref[0], preferred_element_type=jnp.float32)
          + jnp.dot(x1, ws1_ref[0], preferred_element_type=jnp.float32))
    h_ref[...] = (mi * jax.nn.sigmoid(mi)) * sw
    o = (jnp.dot(h_ref[:, :FH], wo0_ref[0], preferred_element_type=jnp.float32)
         + jnp.dot(h_ref[:, FH:], wo1_ref[0],
                   preferred_element_type=jnp.float32))

    for g in range(G):
        t = idx_ref[e * G + g]
        pltpu.make_async_copy(
            out_ref.at[pl.ds(t, 1), :], racc_ref.at[pl.ds(g, 1), :], rd_sem
        ).wait()
    racc_ref[...] += o
    for g in range(G):
        t = idx_ref[e * G + g]
        pltpu.make_async_copy(
            racc_ref.at[pl.ds(g, 1), :], out_ref.at[pl.ds(t, 1), :], wr_sem
        ).start()

    @pl.when(e == E - 1)
    def _drain():
        for g in range(G):
            t = idx_ref[e * G + g]
            pltpu.make_async_copy(
                racc_ref.at[pl.ds(g, 1), :], out_ref.at[pl.ds(t, 1), :], wr_sem
            ).wait()


def _sc_route_body(scores_hbm, x_hbm, idx_out, gate_out, routed_out,
                   row_v, hist_v, csum_v, ge_v, cpos_v, ckey_v, sel_v, gsc_v,
                   rows_v, sem, *, A, E, G, EPW):
    CHUNKS = A // _L
    NBV = _NB // _L
    lanes = lax.iota(jnp.int32, _L)
    ones = jnp.ones((_L,), jnp.int32)
    lane0 = lanes == 0

    def splat(s):
        return jnp.full((_L,), s, jnp.int32)

    def scal(v):
        return jnp.max(v)

    z16 = jnp.zeros((_L,), jnp.int32)
    alltrue = jnp.full((_L,), True)

    def cnt_splat(m):
        return jnp.broadcast_to(plsc.all_reduce_population_count(m), (_L,))

    wid = lax.axis_index("s") * 2 + lax.axis_index("c")

    for ei in range(EPW):
        e = wid * EPW + ei

        pltpu.sync_copy(scores_hbm.at[e], row_v)

        def zero_hist():
            @plsc.parallel_loop(0, _L * NBV, unroll=8)
            def _(i):
                hist_v[i // NBV, pl.ds((i % NBV) * _L, _L)] = z16

        def scan_and_find(target):
            def rb(v, c):
                acc = jnp.zeros((_L,), jnp.int32)
                for r in range(_L):
                    acc = acc + hist_v[r, pl.ds(v * _L, _L)]
                csum_v[pl.ds(v * _L, _L)] = acc
                return c
            lax.fori_loop(0, NBV, rb, 0)

            def sb(j, carry):
                v = NBV - 1 - j
                col = csum_v[pl.ds(v * _L, _L)]
                rsuf = plsc.cumsum(lax.rev(col, (0,))) + carry
                ge_v[pl.ds(v * _L, _L)] = lax.rev(rsuf, (0,))
                return scal(rsuf)
            lax.fori_loop(0, NBV, sb, jnp.int32(0))

            def fb(j, carry):
                found, bstar = carry
                v = NBV - 1 - j
                gvr = lax.rev(ge_v[pl.ds(v * _L, _L)], (0,))
                m = gvr >= target
                has = scal(plsc.all_reduce_population_count(m)) > 0
                k = scal(plsc.all_reduce_ffs(m))
                cand_b = v * _L + (_L - 1 - k)
                hit = jnp.logical_and(jnp.logical_not(found), has)
                return (jnp.logical_or(found, has),
                        jnp.where(hit, cand_b, bstar))
            _, bstar = lax.fori_loop(0, NBV, fb, (False, jnp.int32(0)))

            cnt_b = scal(plsc.load_gather(csum_v, [splat(bstar)]))
            ge_b = scal(plsc.load_gather(ge_v, [splat(bstar)]))
            return bstar, ge_b - cnt_b

        zero_hist()

        @plsc.parallel_loop(0, CHUNKS, unroll=4)
        def _(i):
            s = row_v[pl.ds(i * _L, _L)]
            u = plsc.bitcast(s, jnp.int32)
            key = jnp.where(u < 0, u ^ jnp.int32(0x7FFFFFFF), u)
            row_v[pl.ds(i * _L, _L)] = plsc.bitcast(key, jnp.float32)
            b = (key >> 23) + _NB // 2
            plsc.addupdate_scatter(hist_v, [lanes, b], ones)

        bstar, c_above = scan_and_find(G)
        need = G - c_above

        def px(i, carry):
            ogt, oeq = carry
            key = plsc.bitcast(row_v[pl.ds(i * _L, _L)], jnp.int32)
            b = (key >> 23) + _NB // 2
            pos = i * _L + lanes
            m_gt = b > bstar
            m_eq = b == bstar
            cgt = plsc.cumsum(jnp.where(m_gt, 1, 0))
            ceq = plsc.cumsum(jnp.where(m_eq, 1, 0))
            plsc.store_scatter(sel_v, [ogt + cgt - 1], pos, mask=m_gt)
            plsc.store_scatter(cpos_v, [oeq + ceq - 1], pos, mask=m_eq)
            plsc.store_scatter(ckey_v, [oeq + ceq - 1], key, mask=m_eq)
            return ogt + cnt_splat(m_gt), oeq + cnt_splat(m_eq)
        _, oeq_v = plsc.parallel_loop(0, CHUNKS, carry=(z16, z16))(px)
        m_cnt = scal(oeq_v)

        plsc.store_scatter(ckey_v, [m_cnt + lanes], splat(_IMIN), mask=alltrue)

        zero_hist()
        nv = (m_cnt + _L - 1) // _L

        def pa2(v, c):
            k = ckey_v[pl.ds(v * _L, _L)]
            valid = (v * _L + lanes) < m_cnt
            b2 = (k >> 14) & (_NB - 1)
            plsc.addupdate_scatter(hist_v, [lanes, b2], ones, mask=valid)
            return c
        lax.fori_loop(0, nv, pa2, 0)

        b2star, c_above2 = scan_and_find(need)
        need2 = need - c_above2
        base2 = c_above + c_above2

        def px2(v, carry):
            osel, oeq = carry
            k = ckey_v[pl.ds(v * _L, _L)]
            p = cpos_v[pl.ds(v * _L, _L)]
            valid = (v * _L + lanes) < m_cnt
            b2 = (k >> 14) & (_NB - 1)
            m_gt = jnp.logical_and(b2 > b2star, valid)
            m_eq = jnp.logical_and(b2 == b2star, valid)
            cgt = plsc.cumsum(jnp.where(m_gt, 1, 0))
            ceq = plsc.cumsum(jnp.where(m_eq, 1, 0))
            plsc.store_scatter(sel_v, [osel + cgt - 1], p, mask=m_gt)
            plsc.store_scatter(cpos_v, [oeq + ceq - 1], p, mask=m_eq)
            plsc.store_scatter(ckey_v, [oeq + ceq - 1], k, mask=m_eq)
            return osel + cnt_splat(m_gt), oeq + cnt_splat(m_eq)
        _, oeq2_v = lax.fori_loop(0, nv, px2, (splat(c_above), z16))
        m2 = scal(oeq2_v)

        plsc.store_scatter(ckey_v, [m2 + lanes], splat(_IMIN), mask=alltrue)

        nv2 = (m2 + _L - 1) // _L

        def pick(r, c):
            def mx1(v, mm):
                return jnp.maximum(mm, scal(ckey_v[pl.ds(v * _L, _L)]))
            mx = lax.fori_loop(0, nv2, mx1, jnp.int32(_IMIN))

            def fpos(v, carry):
                found, p = carry
                ck = ckey_v[pl.ds(v * _L, _L)]
                m = ck == mx
                has = scal(plsc.all_reduce_population_count(m)) > 0
                k = scal(plsc.all_reduce_ffs(m))
                hit = jnp.logical_and(jnp.logical_not(found), has)
                return (jnp.logical_or(found, has),
                        jnp.where(hit, v * _L + k, p))
            _, p = lax.fori_loop(0, nv2, fpos, (False, jnp.int32(0)))

            tokp = plsc.load_gather(cpos_v, [splat(p)])
            plsc.store_scatter(sel_v, [splat(base2 + r)], tokp, mask=lane0)
            plsc.store_scatter(ckey_v, [splat(p)], splat(_IMIN), mask=lane0)
            return c
        lax.fori_loop(0, need2, pick, 0)

        for v in range(G // _L):
            sidx = sel_v[pl.ds(v * _L, _L)]
            key = plsc.bitcast(plsc.load_gather(row_v, [sidx]), jnp.int32)
            u = jnp.where(key < 0, key ^ jnp.int32(0x7FFFFFFF), key)
            s = plsc.bitcast(u, jnp.float32)
            gsc_v[pl.ds(v * _L, _L)] = 1.0 / (1.0 + jnp.exp(-s))

        pltpu.sync_copy(sel_v, idx_out.at[e])
        pltpu.sync_copy(gsc_v, gate_out.at[e])

        pltpu.async_copy(x_hbm.at[sel_v], rows_v, sem).wait()
        pltpu.sync_copy(rows_v, routed_out.at[pl.ds(e * G, G)])


def kernel(x, router_DE, moe_w_in_eD_F, moe_w_swiglu_eD_F, moe_w_out_eF_D):
    A, D = x.shape
    E = router_DE.shape[1]
    F = moe_w_in_eD_F.shape[1]
    G = int(A * 0.25 / E)
    G += (-G) % 8
    G = min(G, A)
    EG = E * G

    ABLK = 2048
    scores = pl.pallas_call(
        _scores_body,
        grid=(A // ABLK,),
        in_specs=[pl.BlockSpec((ABLK, D), lambda i: (i, 0)),
                  pl.BlockSpec((D, E), lambda i: (0, 0))],
        out_specs=pl.BlockSpec((E, ABLK), lambda i: (0, i)),
        out_shape=jax.ShapeDtypeStruct((E, A), jnp.float32),
    )(x, router_DE)

    NW = 32
    EPW = E // NW
    mesh = plsc.VectorSubcoreMesh(core_axis_name="c", subcore_axis_name="s")

    route = functools.partial(
        pl.kernel, mesh=mesh,
        out_type=(jax.ShapeDtypeStruct((E, G), jnp.int32),
                  jax.ShapeDtypeStruct((E, G), jnp.float32),
                  jax.ShapeDtypeStruct((EG, D), jnp.float32)),
        scratch_types=[pltpu.VMEM((A,), jnp.float32),
                       pltpu.VMEM((_L, _NB), jnp.int32),
                       pltpu.VMEM((_NB,), jnp.int32),
                       pltpu.VMEM((_NB,), jnp.int32),
                       pltpu.VMEM((A,), jnp.int32),
                       pltpu.VMEM((A + _L,), jnp.int32),
                       pltpu.VMEM((G,), jnp.int32),
                       pltpu.VMEM((G,), jnp.float32),
                       pltpu.VMEM((G, D), jnp.float32),
                       pltpu.SemaphoreType.DMA],
        compiler_params=pltpu.CompilerParams(needs_layout_passes=False),
    )(functools.partial(_sc_route_body, A=A, E=E, G=G, EPW=EPW))
    idx_EG, gates_EG, routed = route(scores, x)

    flat_idx = idx_EG.reshape(-1)
    gates3 = gates_EG.reshape(E, 1, G)

    ZBLK = 2048
    out0 = pl.pallas_call(
        _zero_body,
        grid=(A // ZBLK,),
        out_specs=pl.BlockSpec((ZBLK, D), lambda i: (i, 0)),
        out_shape=jax.ShapeDtypeStruct((A, D), jnp.float32),
    )()

    DH, FH = D // 2, F // 2
    wi3 = moe_w_in_eD_F.reshape(2 * E, DH, F)
    ws3 = moe_w_swiglu_eD_F.reshape(2 * E, DH, F)
    wo3 = moe_w_out_eF_D.reshape(2 * E, FH, D)
    out = pl.pallas_call(
        functools.partial(_ffn_body, A=A, G=G, E=E),
        grid=(E,),
        in_specs=[
            pl.BlockSpec(memory_space=pl.ANY),
            pl.BlockSpec(memory_space=pltpu.SMEM),
            pl.BlockSpec((1, G, D), lambda e: (e, 0, 0)),
            pl.BlockSpec((1, 1, G), lambda e: (e, 0, 0)),
            pl.BlockSpec((1, DH, F), lambda e: (2 * e, 0, 0)),
            pl.BlockSpec((1, DH, F), lambda e: (2 * e + 1, 0, 0)),
            pl.BlockSpec((1, DH, F), lambda e: (2 * e, 0, 0)),
            pl.BlockSpec((1, DH, F), lambda e: (2 * e + 1, 0, 0)),
            pl.BlockSpec((1, FH, D), lambda e: (2 * e, 0, 0)),
            pl.BlockSpec((1, FH, D), lambda e: (2 * e + 1, 0, 0)),
        ],
        out_specs=pl.BlockSpec(memory_space=pl.ANY),
        out_shape=jax.ShapeDtypeStruct((A, D), jnp.float32),
        scratch_shapes=[pltpu.VMEM((G, D), jnp.float32),
                        pltpu.VMEM((G, F), jnp.float32),
                        pltpu.VMEM((G, D), jnp.float32),
                        pltpu.SemaphoreType.DMA,
                        pltpu.SemaphoreType.DMA],
        input_output_aliases={0: 0},
    )(out0, flat_idx, routed.reshape(E, G, D), gates3,
      wi3, wi3, ws3, ws3, wo3, wo3)
    return out
